# baseline (speedup 1.0000x reference)
import functools

import jax
import jax.numpy as jnp
from jax.experimental import pallas as pl
from jax.experimental.pallas import tpu as pltpu
ALPHA = 0.2


def _proj_kernel(x_ref, w_ref, a1_ref, a2r_ref,
                 whb_ref, u1_ref, v1_ref, u2_ref, v2_ref):
    n, d = x_ref.shape
    wh = jnp.dot(x_ref[...], w_ref[...], preferred_element_type=jnp.float32)
    whb_ref[...] = jnp.concatenate(
        [wh, jnp.ones((n, 1), jnp.float32),
         jnp.zeros((n, d - 1), jnp.float32)],
        axis=1).astype(jnp.bfloat16)
    s1 = jnp.dot(wh, a1_ref[...], preferred_element_type=jnp.float32)
    s2 = jax.lax.dot_general(
        a2r_ref[...], wh, (((1,), (1,)), ((), ())),
        preferred_element_type=jnp.float32)
    u1_ref[...] = jnp.exp2(s1)
    v1_ref[...] = jnp.exp2(ALPHA * s1)
    u2_ref[...] = jnp.exp2(s2)
    v2_ref[...] = jnp.exp2(ALPHA * s2)


def _attn_kernel(u1_ref, v1_ref, u2_ref, v2_ref, adj_ref, wh_ref, out_ref,
                 *, d):
    p = jnp.maximum(u1_ref[...] * u2_ref[...],
                    v1_ref[...] * v2_ref[...]) * adj_ref[...]
    acc = jnp.dot(p.astype(jnp.bfloat16), wh_ref[...],
                  preferred_element_type=jnp.float32)
    out_ref[...] = jnp.maximum(acc[:, :d] / acc[:, d:d + 1], 0.0)


def _gat(inputs, adj_rows, W, a1, a2r):
    n, d = inputs.shape
    rows = adj_rows.shape[0]

    wh, u1, v1, u2, v2 = pl.pallas_call(
        _proj_kernel,
        out_shape=[
            jax.ShapeDtypeStruct((n, 2 * d), jnp.bfloat16),
            jax.ShapeDtypeStruct((n, 1), jnp.float32),
            jax.ShapeDtypeStruct((n, 1), jnp.float32),
            jax.ShapeDtypeStruct((1, n), jnp.float32),
            jax.ShapeDtypeStruct((1, n), jnp.float32),
        ],
    )(inputs, W, a1, a2r)

    block_r = max(b for b in (200, 8) if rows % b == 0)
    block_r = min(block_r, rows)
    return pl.pallas_call(
        functools.partial(_attn_kernel, d=d),
        grid=(rows // block_r,),
        in_specs=[
            pl.BlockSpec((block_r, 1), lambda i: (i, 0)),
            pl.BlockSpec((block_r, 1), lambda i: (i, 0)),
            pl.BlockSpec((1, n), lambda i: (0, 0)),
            pl.BlockSpec((1, n), lambda i: (0, 0)),
            pl.BlockSpec((block_r, n), lambda i: (i, 0)),
            pl.BlockSpec((n, 2 * d), lambda i: (0, 0)),
        ],
        out_specs=pl.BlockSpec((block_r, d), lambda i: (i, 0)),
        out_shape=jax.ShapeDtypeStruct((rows, d), jnp.float32),
        compiler_params=pltpu.CompilerParams(
            dimension_semantics=("parallel",)),
    )(u1, v1, u2, v2, adj_rows, wh)


def kernel(inputs, adj, W, a, cmt_weight):
    n, d = inputs.shape
    inv_ln2 = 1.4426950408889634
    a1 = a[:d] * inv_ln2
    a2r = a[d:].reshape(1, d) * inv_ln2

    return _gat(inputs, adj, W, a1, a2r)


# speedup vs baseline: 1.1194x; 1.1194x over previous
import functools

import jax
import jax.numpy as jnp
from jax.experimental import pallas as pl
from jax.experimental.pallas import tpu as pltpu

ALPHA = 0.2

BLOCK_R = 400


def _gat_kernel(x_ref, w_ref, a1_ref, a2r_ref, adj_ref, out_ref,
                wh_s, u1_s, v1_s, u2_s, v2_s, *, d, block_r):
    i = pl.program_id(0)

    @pl.when(i == 0)
    def _proj():
        n = x_ref.shape[0]
        wh = jnp.dot(x_ref[...], w_ref[...],
                     preferred_element_type=jnp.float32)
        wh_s[...] = jnp.concatenate(
            [wh, jnp.ones((n, 1), jnp.float32),
             jnp.zeros((n, d - 1), jnp.float32)],
            axis=1).astype(jnp.bfloat16)
        s1 = jnp.dot(wh, a1_ref[...], preferred_element_type=jnp.float32)
        s2 = jax.lax.dot_general(
            a2r_ref[...], wh, (((1,), (1,)), ((), ())),
            preferred_element_type=jnp.float32)
        u1_s[...] = jnp.exp2(s1)
        v1_s[...] = jnp.exp2(ALPHA * s1)
        u2_s[...] = jnp.exp2(s2)
        v2_s[...] = jnp.exp2(ALPHA * s2)

    r0 = i * block_r
    p = jnp.maximum(u1_s[pl.ds(r0, block_r), :] * u2_s[...],
                    v1_s[pl.ds(r0, block_r), :] * v2_s[...]) * adj_ref[...]
    acc = jnp.dot(p.astype(jnp.bfloat16), wh_s[...],
                  preferred_element_type=jnp.float32)
    out_ref[...] = jnp.maximum(acc[:, :d] / acc[:, d:d + 1], 0.0)


def kernel(inputs, adj, W, a, cmt_weight):
    n, d = inputs.shape
    inv_ln2 = 1.4426950408889634
    a1 = a[:d] * inv_ln2
    a2r = a[d:].reshape(1, d) * inv_ln2

    block_r = max(b for b in (BLOCK_R, 200, 8) if n % b == 0)
    block_r = min(block_r, n)
    return pl.pallas_call(
        functools.partial(_gat_kernel, d=d, block_r=block_r),
        grid=(n // block_r,),
        in_specs=[
            pl.BlockSpec((n, d), lambda i: (0, 0)),
            pl.BlockSpec((d, d), lambda i: (0, 0)),
            pl.BlockSpec((d, 1), lambda i: (0, 0)),
            pl.BlockSpec((1, d), lambda i: (0, 0)),
            pl.BlockSpec((block_r, n), lambda i: (i, 0)),
        ],
        out_specs=pl.BlockSpec((block_r, d), lambda i: (i, 0)),
        out_shape=jax.ShapeDtypeStruct((n, d), jnp.float32),
        scratch_shapes=[
            pltpu.VMEM((n, 2 * d), jnp.bfloat16),
            pltpu.VMEM((n, 1), jnp.float32),
            pltpu.VMEM((n, 1), jnp.float32),
            pltpu.VMEM((1, n), jnp.float32),
            pltpu.VMEM((1, n), jnp.float32),
        ],
        compiler_params=pltpu.CompilerParams(
            dimension_semantics=("arbitrary",)),
    )(inputs, W, a1, a2r, adj)
